# baseline (speedup 1.0000x reference)
import functools

import jax
import jax.numpy as jnp
from jax import lax
from jax.experimental import pallas as pl
from jax.experimental.pallas import tpu as pltpu
from jax.experimental.pallas import tpu_sc as plsc


def _sc_gather(x, emb_table):
    B = x.shape[0]
    V, D = emb_table.shape
    info = plsc.get_sparse_core_info()
    nw = info.num_cores * info.num_subcores
    b_per_w = B // nw
    mesh = plsc.VectorSubcoreMesh(core_axis_name="c", subcore_axis_name="s")

    @functools.partial(
        pl.kernel,
        mesh=mesh,
        out_type=jax.ShapeDtypeStruct((B, D), jnp.float32),
        scratch_types=[
            pltpu.VMEM((b_per_w,), jnp.int32),
            pltpu.VMEM((b_per_w, D), jnp.float32),
            pltpu.SemaphoreType.DMA,
        ],
    )
    def gather_kernel(table_hbm, idx_hbm, out_hbm, idx_v, rows_v, sem):
        wid = lax.axis_index("s") * info.num_cores + lax.axis_index("c")
        base = wid * b_per_w
        pltpu.sync_copy(idx_hbm.at[pl.ds(base, b_per_w)], idx_v)
        pltpu.async_copy(table_hbm.at[idx_v], rows_v, sem).wait()
        pltpu.sync_copy(rows_v, out_hbm.at[pl.ds(base, b_per_w)])

    return gather_kernel(emb_table, x)


def _projection_t(h, wt, b):
    B, D = h.shape
    V = wt.shape[0]
    TV = 2048
    K = 6
    AHEAD = 4
    nv_full = V // TV
    rem = V - nv_full * TV
    nsteps = nv_full + (1 if rem else 0)
    b2 = b.reshape(1, V)

    def body(h_ref, b_ref, w_hbm, o_ref, bufs, wbufs, ht, sems, wsems):
        j = pl.program_id(0)
        slot = lax.rem(j, K)

        @pl.when(j == 0)
        def _():
            ht[...] = lax.transpose(h_ref[...], (1, 0))

        def out_half(step, slot_, half, full):
            n = (TV if full else rem) // 2
            return pltpu.make_async_copy(
                bufs.at[slot_, pl.ds(half * n, n)],
                o_ref.at[pl.ds(step * TV + half * n, n)],
                sems.at[slot_],
            )

        def w_fetch(step, slot_, full):
            n = TV if full else rem
            return pltpu.make_async_copy(
                w_hbm.at[pl.ds(step * TV, n)],
                wbufs.at[slot_, pl.ds(0, n)],
                wsems.at[slot_],
            )

        @pl.when(j == 0)
        def _():
            for t in range(min(AHEAD, nsteps)):
                w_fetch(t, t % K, t < nv_full).start(priority=t % 2)

        for s in range(K):
            @pl.when(slot == s)
            def _(s=s):
                sf = (s + AHEAD) % K
                jf = j + AHEAD

                @pl.when(jf < nv_full)
                def _():
                    w_fetch(jf, sf, True).start(priority=sf % 2)

                if rem:
                    @pl.when(jf == nv_full)
                    def _():
                        w_fetch(jf, sf, False).start(priority=sf % 2)

        for s in range(K):
            @pl.when((j >= K) & (slot == s))
            def _(s=s):
                out_half(j - K, s, 0, True).wait()
                out_half(j - K, s, 1, True).wait()

        bt = lax.transpose(b_ref[...], (1, 0))

        for s in range(K):
            @pl.when(slot == s)
            def _(s=s):
                if rem:
                    @pl.when(j == nv_full)
                    def _():
                        w_fetch(j, s, False).wait()

                @pl.when(j < nv_full)
                def _():
                    w_fetch(j, s, True).wait()

                bufs[s] = (
                    jnp.dot(wbufs[s], ht[...],
                            preferred_element_type=jnp.float32)
                    + bt
                )

                @pl.when(j < nv_full)
                def _():
                    out_half(j, s, 0, True).start(priority=0)
                    out_half(j, s, 1, True).start(priority=1)

                if rem:
                    @pl.when(j == nv_full)
                    def _():
                        out_half(j, s, 0, False).start(priority=0)
                        out_half(j, s, 1, False).start(priority=1)

        @pl.when(j == nsteps - 1)
        def _():
            for t in range(max(0, nsteps - K), nsteps):
                full = not (rem and t == nv_full)
                out_half(t, t % K, 0, full).wait()
                out_half(t, t % K, 1, full).wait()

    return pl.pallas_call(
        body,
        grid=(nsteps,),
        in_specs=[
            pl.BlockSpec((B, D), lambda j: (0, 0)),
            pl.BlockSpec((1, TV), lambda j: (0, j)),
            pl.BlockSpec(memory_space=pl.ANY),
        ],
        out_specs=pl.BlockSpec(memory_space=pl.ANY),
        out_shape=jax.ShapeDtypeStruct((V, B), jnp.float32),
        scratch_shapes=[
            pltpu.VMEM((K, TV, B), jnp.float32),
            pltpu.VMEM((K, TV, D), jnp.float32),
            pltpu.VMEM((D, B), jnp.float32),
            pltpu.SemaphoreType.DMA((K,)),
            pltpu.SemaphoreType.DMA((K,)),
        ],
    )(h, b2, wt)


def kernel(x, emb_table, W, b):
    h = _sc_gather(x, emb_table)
    out_t = _projection_t(h, W.T, b)
    return out_t.T


# speedup vs baseline: 1.0033x; 1.0033x over previous
import functools

import jax
import jax.numpy as jnp
from jax import lax
from jax.experimental import pallas as pl
from jax.experimental.pallas import tpu as pltpu
from jax.experimental.pallas import tpu_sc as plsc


def _sc_gather(x, emb_table):
    B = x.shape[0]
    V, D = emb_table.shape
    info = plsc.get_sparse_core_info()
    nw = info.num_cores * info.num_subcores
    b_per_w = B // nw
    mesh = plsc.VectorSubcoreMesh(core_axis_name="c", subcore_axis_name="s")

    @functools.partial(
        pl.kernel,
        mesh=mesh,
        out_type=jax.ShapeDtypeStruct((B, D), jnp.float32),
        scratch_types=[
            pltpu.VMEM((b_per_w,), jnp.int32),
            pltpu.VMEM((b_per_w, D), jnp.float32),
            pltpu.SemaphoreType.DMA,
        ],
    )
    def gather_kernel(table_hbm, idx_hbm, out_hbm, idx_v, rows_v, sem):
        wid = lax.axis_index("s") * info.num_cores + lax.axis_index("c")
        base = wid * b_per_w
        pltpu.sync_copy(idx_hbm.at[pl.ds(base, b_per_w)], idx_v)
        pltpu.async_copy(table_hbm.at[idx_v], rows_v, sem).wait()
        pltpu.sync_copy(rows_v, out_hbm.at[pl.ds(base, b_per_w)])

    return gather_kernel(emb_table, x)


def _projection_t(h, wt, b):
    B, D = h.shape
    V = wt.shape[0]
    TV = 2048
    K = 6
    AHEAD = 5
    nv_full = V // TV
    rem = V - nv_full * TV
    nsteps = nv_full + (1 if rem else 0)
    b2 = b.reshape(1, V)

    def body(h_ref, b_ref, w_hbm, o_ref, bufs, wbufs, ht, sems, wsems):
        j = pl.program_id(0)
        slot = lax.rem(j, K)

        @pl.when(j == 0)
        def _():
            ht[...] = lax.transpose(h_ref[...], (1, 0))

        def out_copy(step, slot_, full):
            n = TV if full else rem
            return pltpu.make_async_copy(
                bufs.at[slot_, pl.ds(0, n)],
                o_ref.at[pl.ds(step * TV, n)],
                sems.at[slot_],
            )

        def w_fetch(step, slot_, full):
            n = TV if full else rem
            return pltpu.make_async_copy(
                w_hbm.at[pl.ds(step * TV, n)],
                wbufs.at[slot_, pl.ds(0, n)],
                wsems.at[slot_],
            )

        @pl.when(j == 0)
        def _():
            for t in range(min(AHEAD, nsteps)):
                w_fetch(t, t % K, t < nv_full).start(priority=t % 2)

        for s in range(K):
            @pl.when(slot == s)
            def _(s=s):
                sf = (s + AHEAD) % K
                jf = j + AHEAD

                @pl.when(jf < nv_full)
                def _():
                    w_fetch(jf, sf, True).start(priority=sf % 2)

                if rem:
                    @pl.when(jf == nv_full)
                    def _():
                        w_fetch(jf, sf, False).start(priority=sf % 2)

        for s in range(K):
            @pl.when((j >= K) & (slot == s))
            def _(s=s):
                out_copy(j - K, s, True).wait()

        bt = lax.transpose(b_ref[...], (1, 0))

        for s in range(K):
            @pl.when(slot == s)
            def _(s=s):
                if rem:
                    @pl.when(j == nv_full)
                    def _():
                        w_fetch(j, s, False).wait()

                @pl.when(j < nv_full)
                def _():
                    w_fetch(j, s, True).wait()

                bufs[s] = (
                    jnp.dot(wbufs[s], ht[...],
                            preferred_element_type=jnp.float32)
                    + bt
                )

                @pl.when(j < nv_full)
                def _():
                    out_copy(j, s, True).start(priority=s % 2)

                if rem:
                    @pl.when(j == nv_full)
                    def _():
                        out_copy(j, s, False).start(priority=s % 2)

        @pl.when(j == nsteps - 1)
        def _():
            for t in range(max(0, nsteps - K), nsteps):
                out_copy(t, t % K, not (rem and t == nv_full)).wait()

    return pl.pallas_call(
        body,
        grid=(nsteps,),
        in_specs=[
            pl.BlockSpec((B, D), lambda j: (0, 0)),
            pl.BlockSpec((1, TV), lambda j: (0, j)),
            pl.BlockSpec(memory_space=pl.ANY),
        ],
        out_specs=pl.BlockSpec(memory_space=pl.ANY),
        out_shape=jax.ShapeDtypeStruct((V, B), jnp.float32),
        scratch_shapes=[
            pltpu.VMEM((K, TV, B), jnp.float32),
            pltpu.VMEM((K, TV, D), jnp.float32),
            pltpu.VMEM((D, B), jnp.float32),
            pltpu.SemaphoreType.DMA((K,)),
            pltpu.SemaphoreType.DMA((K,)),
        ],
    )(h, b2, wt)


def kernel(x, emb_table, W, b):
    h = _sc_gather(x, emb_table)
    out_t = _projection_t(h, W.T, b)
    return out_t.T
